# baseline (speedup 1.0000x reference)
import functools

import jax
import jax.numpy as jnp
from jax import lax
from jax.experimental import pallas as pl
from jax.experimental.pallas import tpu as pltpu
from jax.experimental.pallas import tpu_sc as plsc

NC = 2
NS = 16
NW = NC * NS

B = 4096 * 200
D = 128
BPW = B // NW

V = 10240
VPS = V // NS

CH = 80
NCHW = BPW // CH
NP = NCHW // 2
IBLK = 32
IBW = IBLK * CH
NIB = NCHW // IBLK

_mesh = plsc.VectorSubcoreMesh(core_axis_name="c", subcore_axis_name="s")


@functools.partial(
    pl.kernel,
    out_type=jax.ShapeDtypeStruct((B, D), jnp.float32),
    mesh=_mesh,
    scratch_types=[
        pltpu.VMEM_SHARED((V, D), jnp.float32),
        pltpu.VMEM((2 * IBW,), jnp.int32),
        pltpu.VMEM((4, CH, D), jnp.float32),
        pltpu.SemaphoreType.DMA,
        pltpu.SemaphoreType.DMA,
        pltpu.SemaphoreType.DMA,
        pltpu.SemaphoreType.DMA,
    ],
)
def _gather_kernel(x_hbm, tab_hbm, out_hbm, tab_s, idx_v, rows_v,
                   isem, gsem, hsem, wsem):
    cid = lax.axis_index("c")
    sid = lax.axis_index("s")
    wid = sid * NC + cid

    pltpu.async_copy(x_hbm.at[wid, pl.ds(0, IBW)], idx_v.at[pl.ds(0, IBW)], isem)
    pltpu.sync_copy(tab_hbm.at[pl.ds(sid * VPS, VPS)],
                    tab_s.at[pl.ds(sid * VPS, VPS)])
    plsc.subcore_barrier()

    base = wid * BPW

    def step(p, carry):
        @pl.when(p < NP)
        def _():
            c0 = 2 * p
            b0 = c0 % 4
            b1 = (c0 + 1) % 4
            blk = c0 // IBLK
            ib = blk % 2

            @pl.when(c0 % IBLK == 0)
            def _():
                pltpu.make_async_copy(
                    x_hbm.at[wid, pl.ds(0, IBW)],
                    idx_v.at[pl.ds(ib * IBW, IBW)], isem).wait()
                @pl.when(blk + 1 < NIB)
                def _():
                    pltpu.async_copy(
                        x_hbm.at[wid, pl.ds((blk + 1) * IBW, IBW)],
                        idx_v.at[pl.ds(((blk + 1) % 2) * IBW, IBW)], isem)

            @pl.when(p >= 2)
            def _():
                pltpu.make_async_copy(
                    rows_v.at[b0], out_hbm.at[pl.ds(base, CH)], wsem).wait()
                pltpu.make_async_copy(
                    rows_v.at[b1], out_hbm.at[pl.ds(base, CH)], wsem).wait()

            o = ib * IBW + (c0 % IBLK) * CH
            pltpu.async_copy(tab_s.at[idx_v.at[pl.ds(o, CH)]],
                             rows_v.at[b0], gsem)
            pltpu.async_copy(tab_s.at[idx_v.at[pl.ds(o + CH, CH)]],
                             rows_v.at[b1], gsem)

        @pl.when(p >= 1)
        def _():
            c0 = 2 * (p - 1)
            b0 = c0 % 4
            b1 = (c0 + 1) % 4
            pltpu.make_async_copy(tab_s.at[pl.ds(0, CH)],
                                  rows_v.at[b0], gsem).wait()
            pltpu.async_copy(rows_v.at[b0],
                             out_hbm.at[pl.ds(base + c0 * CH, CH)], wsem)
            pltpu.make_async_copy(tab_s.at[pl.ds(0, CH)],
                                  rows_v.at[b1], gsem).wait()
            pltpu.async_copy(rows_v.at[b1],
                             out_hbm.at[pl.ds(base + (c0 + 1) * CH, CH)], wsem)
        return carry

    lax.fori_loop(0, NP + 1, step, 0)

    for b in range(4):
        pltpu.make_async_copy(rows_v.at[b], out_hbm.at[pl.ds(base, CH)],
                              wsem).wait()


def kernel(x, embedding):
    xw = x.reshape(NW, BPW).astype(jnp.int32)
    tab = jnp.pad(embedding, ((0, V - embedding.shape[0]), (0, 0)))
    out = _gather_kernel(xw, tab)
    return out.reshape(x.shape[0], x.shape[1], D)


# speedup vs baseline: 1.0295x; 1.0295x over previous
import functools

import jax
import jax.numpy as jnp
from jax import lax
from jax.experimental import pallas as pl
from jax.experimental.pallas import tpu as pltpu
from jax.experimental.pallas import tpu_sc as plsc

NC = 2
NS = 16
NW = NC * NS

B = 4096 * 200
D = 128
BPW = B // NW

V = 10000
VPS = 632

CH = 80
NCHW = BPW // CH
NP = NCHW // 2
IBLK = 32
IBW = IBLK * CH
NIB = NCHW // IBLK

_mesh = plsc.VectorSubcoreMesh(core_axis_name="c", subcore_axis_name="s")


@functools.partial(
    pl.kernel,
    out_type=jax.ShapeDtypeStruct((B, D), jnp.float32),
    mesh=_mesh,
    scratch_types=[
        pltpu.VMEM_SHARED((V, D), jnp.float32),
        pltpu.VMEM((2 * IBW,), jnp.int32),
        pltpu.VMEM((4, CH, D), jnp.float32),
        pltpu.SemaphoreType.DMA,
        pltpu.SemaphoreType.DMA,
        pltpu.SemaphoreType.DMA,
        pltpu.SemaphoreType.DMA,
    ],
)
def _gather_kernel(x_hbm, tab_hbm, out_hbm, tab_s, idx_v, rows_v,
                   isem, gsem, hsem, wsem):
    cid = lax.axis_index("c")
    sid = lax.axis_index("s")
    wid = sid * NC + cid

    pltpu.async_copy(x_hbm.at[wid, pl.ds(0, IBW)], idx_v.at[pl.ds(0, IBW)], isem)
    @pl.when(sid < NS - 1)
    def _():
        pltpu.sync_copy(tab_hbm.at[pl.ds(sid * VPS, VPS)],
                        tab_s.at[pl.ds(sid * VPS, VPS)])
    @pl.when(sid == NS - 1)
    def _():
        pltpu.sync_copy(tab_hbm.at[pl.ds((NS - 1) * VPS, V - (NS - 1) * VPS)],
                        tab_s.at[pl.ds((NS - 1) * VPS, V - (NS - 1) * VPS)])
    plsc.subcore_barrier()

    base = wid * BPW

    def step(p, carry):
        @pl.when(p < NP)
        def _():
            c0 = 2 * p
            b0 = c0 % 4
            b1 = (c0 + 1) % 4
            blk = c0 // IBLK
            ib = blk % 2

            @pl.when(c0 % IBLK == 0)
            def _():
                pltpu.make_async_copy(
                    x_hbm.at[wid, pl.ds(0, IBW)],
                    idx_v.at[pl.ds(ib * IBW, IBW)], isem).wait()
                @pl.when(blk + 1 < NIB)
                def _():
                    pltpu.async_copy(
                        x_hbm.at[wid, pl.ds((blk + 1) * IBW, IBW)],
                        idx_v.at[pl.ds(((blk + 1) % 2) * IBW, IBW)], isem)

            @pl.when(p >= 2)
            def _():
                pltpu.make_async_copy(
                    rows_v.at[b0], out_hbm.at[pl.ds(base, CH)], wsem).wait()
                pltpu.make_async_copy(
                    rows_v.at[b1], out_hbm.at[pl.ds(base, CH)], wsem).wait()

            o = ib * IBW + (c0 % IBLK) * CH
            pltpu.async_copy(tab_s.at[idx_v.at[pl.ds(o, CH)]],
                             rows_v.at[b0], gsem)
            pltpu.async_copy(tab_s.at[idx_v.at[pl.ds(o + CH, CH)]],
                             rows_v.at[b1], gsem)

        @pl.when(p >= 1)
        def _():
            c0 = 2 * (p - 1)
            b0 = c0 % 4
            b1 = (c0 + 1) % 4
            pltpu.make_async_copy(tab_s.at[pl.ds(0, CH)],
                                  rows_v.at[b0], gsem).wait()
            pltpu.async_copy(rows_v.at[b0],
                             out_hbm.at[pl.ds(base + c0 * CH, CH)], wsem)
            pltpu.make_async_copy(tab_s.at[pl.ds(0, CH)],
                                  rows_v.at[b1], gsem).wait()
            pltpu.async_copy(rows_v.at[b1],
                             out_hbm.at[pl.ds(base + (c0 + 1) * CH, CH)], wsem)
        return carry

    lax.fori_loop(0, NP + 1, step, 0)

    for b in range(4):
        pltpu.make_async_copy(rows_v.at[b], out_hbm.at[pl.ds(base, CH)],
                              wsem).wait()


def kernel(x, embedding):
    xw = x.reshape(NW, BPW).astype(jnp.int32)
    out = _gather_kernel(xw, embedding)
    return out.reshape(x.shape[0], x.shape[1], D)
